# baseline (speedup 1.0000x reference)
import functools

import jax
import jax.numpy as jnp
from jax import lax
from jax.experimental import pallas as pl
from jax.experimental.pallas import tpu as pltpu
from jax.experimental.pallas import tpu_sc as plsc

B, Q, D = 64, 8, 1024
E, F = 8, 2048
TOP_K = 2
T = B * Q
FB = 2048
NF = F // FB

_NC, _NS = 2, 16
_NW = _NC * _NS
_TPW = T // _NW


def _logits_body(x_ref, gw_ref, out_ref):
    out_ref[...] = jax.lax.dot_general(
        gw_ref[...], x_ref[...], (((1,), (1,)), ((), ())),
        preferred_element_type=jnp.float32)


def _logits(x, gate_w):
    return pl.pallas_call(
        _logits_body,
        out_shape=jax.ShapeDtypeStruct((E, T), jnp.float32),
    )(x, gate_w)


def _gate_body(logits_hbm, comb_hbm, buf, obuf):
    wid = lax.axis_index("s") * _NC + lax.axis_index("c")
    base = wid * _TPW * E
    pltpu.sync_copy(logits_hbm.at[pl.ds(base, _TPW * E)], buf)
    p = []
    for e in range(E):
        p.append(buf[pl.ds(e * _TPW, _TPW)])
    m = p[0]
    for e in range(1, E):
        m = jnp.maximum(m, p[e])
    s = jnp.zeros((16,), jnp.float32)
    for e in range(E):
        p[e] = jnp.exp(p[e] - m)
        s = s + p[e]
    for e in range(E):
        p[e] = p[e] / s
    m1 = p[0]
    for e in range(1, E):
        m1 = jnp.maximum(m1, p[e])
    neg = jnp.full((16,), -1.0, jnp.float32)
    m2 = jnp.where(p[0] < m1, p[0], neg)
    for e in range(1, E):
        m2 = jnp.maximum(m2, jnp.where(p[e] < m1, p[e], neg))
    den = m1 + m2
    for e in range(E):
        obuf[pl.ds(e * _TPW, _TPW)] = jnp.where(p[e] >= m2, p[e], 0.0) / den
    pltpu.sync_copy(obuf, comb_hbm.at[pl.ds(base, _TPW * E)])


def _gate(logits_flat):
    mesh = plsc.VectorSubcoreMesh(core_axis_name="c", subcore_axis_name="s")
    return pl.kernel(
        _gate_body,
        mesh=mesh,
        out_type=jax.ShapeDtypeStruct((T * E,), jnp.float32),
        scratch_types=[
            pltpu.VMEM((_TPW * E,), jnp.float32),
            pltpu.VMEM((_TPW * E,), jnp.float32),
        ],
    )(logits_flat)


def _moe_body(x_ref, comb_ref, w1_ref, w3_ref, w2_ref, out_ref, xbf_ref):
    e = pl.program_id(0)
    f = pl.program_id(1)
    is_first = (e == 0) & (f == 0)

    @pl.when(is_first)
    def _prep():
        xbf_ref[...] = x_ref[...].astype(jnp.bfloat16)

    xb = xbf_ref[...]
    w1b = w1_ref[0].astype(jnp.bfloat16)
    w3b = w3_ref[0].astype(jnp.bfloat16)
    w2b = w2_ref[0].astype(jnp.bfloat16)
    h1 = jax.lax.dot_general(xb, w1b, (((1,), (1,)), ((), ())),
                             preferred_element_type=jnp.float32)
    h3 = jax.lax.dot_general(xb, w3b, (((1,), (1,)), ((), ())),
                             preferred_element_type=jnp.float32)
    lane = jax.lax.broadcasted_iota(jnp.int32, (1, E), 1)
    c_col = jnp.sum(jnp.where(lane == e, comb_ref[...], 0.0),
                    axis=1, keepdims=True)
    h1b = h1.astype(jnp.bfloat16)
    h3b = (h3 * c_col).astype(jnp.bfloat16)
    sig = 0.5 * jnp.tanh(0.5 * h1b) + 0.5
    h = (h1b * sig) * h3b
    contrib = jax.lax.dot_general(h, w2b,
                                  (((1,), (1,)), ((), ())),
                                  preferred_element_type=jnp.float32)

    @pl.when(is_first)
    def _init():
        out_ref[...] = contrib

    @pl.when(jnp.logical_not(is_first))
    def _add():
        out_ref[...] += contrib


def _moe(x, comb, w1, w3, w2):
    return pl.pallas_call(
        _moe_body,
        grid=(E, NF),
        in_specs=[
            pl.BlockSpec((T, D), lambda e, f: (0, 0)),
            pl.BlockSpec((T, E), lambda e, f: (0, 0)),
            pl.BlockSpec((1, FB, D), lambda e, f: (e, f, 0)),
            pl.BlockSpec((1, FB, D), lambda e, f: (e, f, 0)),
            pl.BlockSpec((1, D, FB), lambda e, f: (e, 0, f)),
        ],
        out_specs=pl.BlockSpec((T, D), lambda e, f: (0, 0)),
        out_shape=jax.ShapeDtypeStruct((T, D), jnp.float32),
        scratch_shapes=[
            pltpu.VMEM((T, D), jnp.bfloat16),
        ],
    )(x, comb, w1, w3, w2)


@jax.jit
def _fwd(x, gate_w, w1, w3, w2):
    logits_et = _logits(x, gate_w)
    lg_w = logits_et.reshape(E, _NW, _TPW).transpose(1, 0, 2).reshape(T * E)
    comb_w = _gate(lg_w)
    comb = comb_w.reshape(_NW, E, _TPW).transpose(0, 2, 1).reshape(T, E)
    return _moe(x, comb, w1, w3, w2)


def kernel(hidden_states, gate_w, w1, w3, w2):
    orig_shape = hidden_states.shape
    x = hidden_states.reshape(-1, orig_shape[-1])
    out = _fwd(x, gate_w, w1, w3, w2)
    return out.reshape(orig_shape)


# speedup vs baseline: 1.2859x; 1.2859x over previous
import functools

import jax
import jax.numpy as jnp
from jax.experimental import pallas as pl
from jax.experimental.pallas import tpu as pltpu

B, Q, D = 64, 8, 1024
E, F = 8, 2048
TOP_K = 2
T = B * Q
FB = 1024
NF = F // FB


def _moe_body(x_ref, gw_ref, w1_ref, w3_ref, w2_ref, out_ref, comb_ref, xbf_ref):
    e = pl.program_id(0)
    f = pl.program_id(1)
    is_first = (e == 0) & (f == 0)

    @pl.when(is_first)
    def _router():
        x = x_ref[...]
        xbf_ref[...] = x.astype(jnp.bfloat16)
        logits = jax.lax.dot_general(
            x, gw_ref[...], (((1,), (1,)), ((), ())),
            preferred_element_type=jnp.float32)
        m = jnp.max(logits, axis=-1, keepdims=True)
        ex = jnp.exp(logits - m)
        p = ex / jnp.sum(ex, axis=-1, keepdims=True)
        m1 = jnp.max(p, axis=-1, keepdims=True)
        neg = jnp.full_like(p, -1.0)
        m2 = jnp.max(jnp.where(p < m1, p, neg), axis=-1, keepdims=True)
        sel = p >= m2
        comb_ref[...] = jnp.where(sel, p, 0.0) / (m1 + m2)

    xb = xbf_ref[...]
    w1b = w1_ref[0].astype(jnp.bfloat16)
    w3b = w3_ref[0].astype(jnp.bfloat16)
    w2b = w2_ref[0].astype(jnp.bfloat16)
    h1 = jax.lax.dot_general(xb, w1b, (((1,), (1,)), ((), ())),
                             preferred_element_type=jnp.float32)
    h3 = jax.lax.dot_general(xb, w3b, (((1,), (1,)), ((), ())),
                             preferred_element_type=jnp.float32)
    lane = jax.lax.broadcasted_iota(jnp.int32, (1, E), 1)
    c_col = jnp.sum(jnp.where(lane == e, comb_ref[...], 0.0),
                    axis=1, keepdims=True)
    sig = 0.5 * jnp.tanh(0.5 * h1) + 0.5
    h = ((h1 * sig) * (h3 * c_col)).astype(jnp.bfloat16)
    contrib = jax.lax.dot_general(h, w2b,
                                  (((1,), (1,)), ((), ())),
                                  preferred_element_type=jnp.float32)

    @pl.when(is_first)
    def _init():
        out_ref[...] = contrib

    @pl.when(jnp.logical_not(is_first))
    def _add():
        out_ref[...] += contrib


@functools.partial(jax.jit, static_argnums=())
def _moe(x, gate_w, w1, w3, w2):
    return pl.pallas_call(
        _moe_body,
        grid=(E, NF),
        in_specs=[
            pl.BlockSpec((T, D), lambda e, f: (0, 0)),
            pl.BlockSpec((E, D), lambda e, f: (0, 0)),
            pl.BlockSpec((1, FB, D), lambda e, f: (e, f, 0)),
            pl.BlockSpec((1, FB, D), lambda e, f: (e, f, 0)),
            pl.BlockSpec((1, D, FB), lambda e, f: (e, 0, f)),
        ],
        out_specs=pl.BlockSpec((T, D), lambda e, f: (0, 0)),
        out_shape=jax.ShapeDtypeStruct((T, D), jnp.float32),
        scratch_shapes=[
            pltpu.VMEM((T, E), jnp.float32),
            pltpu.VMEM((T, D), jnp.bfloat16),
        ],
    )(x, gate_w, w1, w3, w2)


def kernel(hidden_states, gate_w, w1, w3, w2):
    orig_shape = hidden_states.shape
    x = hidden_states.reshape(-1, orig_shape[-1])
    out = _moe(x, gate_w, w1, w3, w2)
    return out.reshape(orig_shape)
